# baseline (speedup 1.0000x reference)
import functools

import jax
import jax.numpy as jnp
from jax import lax
from jax.experimental import pallas as pl
from jax.experimental.pallas import tpu as pltpu

N_USERS = 100000
N_ITEMS = 100000
EMB = 64
HID = 128
P = 3
B = 4096

K = 16
STEPS = B // K

BB = 512
GRID = B // BB


def _gather_body(*refs):
    row_refs = refs[3:3 + 3 * K]
    uout, pout, nout = refs[3 + 3 * K:]
    for s, out in enumerate((uout, pout, nout)):
        for j in range(K):
            out[pl.ds(j * P, P), :] = row_refs[s * K + j][0]


def _gather(user_table, item_table, u_idx, p_idx, n_idx):
    out = jax.ShapeDtypeStruct((B * P, HID), jnp.float32)

    def row_spec(s, j):
        def imap(i, ui, pi, ni):
            idx = (ui, ui, pi, ni)[s + 1]
            return (idx[i * K + j], 0, 0)
        return pl.BlockSpec((1, P, HID), imap)

    grid_spec = pltpu.PrefetchScalarGridSpec(
        num_scalar_prefetch=3,
        grid=(STEPS,),
        in_specs=[row_spec(0, j) for j in range(K)] +
                 [row_spec(1, j) for j in range(K)] +
                 [row_spec(2, j) for j in range(K)],
        out_specs=[pl.BlockSpec((K * P, HID), lambda i, *_: (i, 0))] * 3,
    )
    return pl.pallas_call(
        _gather_body,
        grid_spec=grid_spec,
        out_shape=(out, out, out),
    )(u_idx, p_idx, n_idx,
      *([user_table] * K), *([item_table] * K), *([item_table] * K))


def _attn_math(PU, PPos, PNeg, a):
    dot = lambda x, y: jax.lax.dot(
        x, y, precision=jax.lax.Precision.HIGHEST,
        preferred_element_type=jnp.float32)
    MU = [dot(PU[k], a) for k in range(P)]

    def max3(v0, v1, v2):
        return jnp.maximum(jnp.maximum(v0, v1), v2)

    def soft3(v):
        m = max3(v[0], v[1], v[2])
        e = [jnp.exp(x - m) for x in v]
        r = 1.0 / (e[0] + e[1] + e[2])
        return [x * r for x in e]

    def pair(PI):
        M = [[jnp.sum(MU[p] * PI[q], axis=1, keepdims=True)
              for q in range(P)] for p in range(P)]
        u_att = soft3([max3(M[p][0], M[p][1], M[p][2]) for p in range(P)])
        i_att = soft3([max3(M[0][q], M[1][q], M[2][q]) for q in range(P)])
        att_u = u_att[0] * PU[0] + u_att[1] * PU[1] + u_att[2] * PU[2]
        att_i = i_att[0] * PI[0] + i_att[1] * PI[1] + i_att[2] * PI[2]
        return att_u, att_i

    pu_att, pi_att = pair(PPos)
    nu_att, ni_att = pair(PNeg)
    return pu_att, pi_att, nu_att, ni_att


def _attn_body(u_ref, p_ref, n_ref, wu_ref, wi_ref, a_ref,
               pu_ref, pi_ref, nu_ref, ni_ref):
    dot = lambda x, y: jax.lax.dot(
        x, y, precision=jax.lax.Precision.HIGHEST,
        preferred_element_type=jnp.float32)
    ZU = dot(u_ref[...], wu_ref[...]).reshape(BB, P, EMB)
    ZP = dot(p_ref[...], wi_ref[...]).reshape(BB, P, EMB)
    ZN = dot(n_ref[...], wi_ref[...]).reshape(BB, P, EMB)
    PU = [ZU[:, k, :] for k in range(P)]
    PPos = [ZP[:, k, :] for k in range(P)]
    PNeg = [ZN[:, k, :] for k in range(P)]
    pu, pi, nu, ni = _attn_math(PU, PPos, PNeg, a_ref[...])
    pu_ref[...] = pu
    pi_ref[...] = pi
    nu_ref[...] = nu
    ni_ref[...] = ni


def _tc_attention(u_g, p_g, n_g, W_u, W_i, A):
    out = jax.ShapeDtypeStruct((B, EMB), jnp.float32)
    row_spec = pl.BlockSpec((BB * P, HID), lambda i: (i, 0))
    full = lambda s: pl.BlockSpec(s, lambda i: (0, 0))
    return pl.pallas_call(
        _attn_body,
        grid=(GRID,),
        in_specs=[row_spec, row_spec, row_spec,
                  full((HID, EMB)), full((HID, EMB)), full((EMB, EMB))],
        out_specs=[pl.BlockSpec((BB, EMB), lambda i: (i, 0))] * 4,
        out_shape=(out, out, out, out),
    )(u_g, p_g, n_g, W_u, W_i, A)


def kernel(users, pos_items, neg_items, multi_user_embed, multi_item_embed,
           W_u, W_i, A):
    u_g, p_g, n_g = _gather(multi_user_embed, multi_item_embed,
                            users.astype(jnp.int32),
                            pos_items.astype(jnp.int32),
                            neg_items.astype(jnp.int32))
    return _tc_attention(u_g, p_g, n_g, W_u, W_i, A)


# speedup vs baseline: 1.3044x; 1.3044x over previous
import functools

import jax
import jax.numpy as jnp
from jax import lax
from jax.experimental import pallas as pl
from jax.experimental.pallas import tpu as pltpu

N_USERS = 100000
N_ITEMS = 100000
EMB = 64
HID = 128
P = 3
B = 4096

BB = 512
GRID = B // BB


def _attn_math(PU, PPos, PNeg, a):
    dot = lambda x, y: jax.lax.dot(
        x, y, precision=jax.lax.Precision.HIGHEST,
        preferred_element_type=jnp.float32)
    MU = [dot(PU[k], a) for k in range(P)]

    def max3(v0, v1, v2):
        return jnp.maximum(jnp.maximum(v0, v1), v2)

    def soft3(v):
        m = max3(v[0], v[1], v[2])
        e = [jnp.exp(x - m) for x in v]
        r = 1.0 / (e[0] + e[1] + e[2])
        return [x * r for x in e]

    def pair(PI):
        M = [[jnp.sum(MU[p] * PI[q], axis=1, keepdims=True)
              for q in range(P)] for p in range(P)]
        u_att = soft3([max3(M[p][0], M[p][1], M[p][2]) for p in range(P)])
        i_att = soft3([max3(M[0][q], M[1][q], M[2][q]) for q in range(P)])
        att_u = u_att[0] * PU[0] + u_att[1] * PU[1] + u_att[2] * PU[2]
        att_i = i_att[0] * PI[0] + i_att[1] * PI[1] + i_att[2] * PI[2]
        return att_u, att_i

    pu_att, pi_att = pair(PPos)
    nu_att, ni_att = pair(PNeg)
    return pu_att, pi_att, nu_att, ni_att


def _fused_body(ui_ref, pi_ref, ni_ref,
                ut_any, it_any, wu_ref, wi_ref, a_ref,
                pu_ref, pi_out_ref, nu_ref, ni_out_ref,
                ubuf, pbuf, nbuf, usem, psem, nsem):
    i = pl.program_id(0)
    base = i * BB

    def issue(idx_ref, table, buf, sem):
        def body(j, prio):
            row = idx_ref[base + j]
            pltpu.make_async_copy(
                table.at[row], buf.at[:, j, :], sem
            ).start(priority=prio)
        UNROLL = 8
        @pl.loop(0, BB, step=UNROLL)
        def _(j0):
            for u in range(UNROLL):
                body(j0 + u, u % 2)

    issue(ui_ref, ut_any, ubuf, usem)
    issue(pi_ref, it_any, pbuf, psem)
    issue(ni_ref, it_any, nbuf, nsem)
    pltpu.make_async_copy(ubuf, ubuf, usem).wait()
    pltpu.make_async_copy(pbuf, pbuf, psem).wait()
    pltpu.make_async_copy(nbuf, nbuf, nsem).wait()

    wu, wi = wu_ref[...], wi_ref[...]
    dot = lambda x, y: jax.lax.dot(
        x, y, precision=jax.lax.Precision.HIGHEST,
        preferred_element_type=jnp.float32)
    PU = [dot(ubuf[k], wu) for k in range(P)]
    PPos = [dot(pbuf[k], wi) for k in range(P)]
    PNeg = [dot(nbuf[k], wi) for k in range(P)]
    pu, pi, nu, ni = _attn_math(PU, PPos, PNeg, a_ref[...])
    pu_ref[...] = pu
    pi_out_ref[...] = pi
    nu_ref[...] = nu
    ni_out_ref[...] = ni


def kernel(users, pos_items, neg_items, multi_user_embed, multi_item_embed,
           W_u, W_i, A):
    out = jax.ShapeDtypeStruct((B, EMB), jnp.float32)
    full = lambda s: pl.BlockSpec(s, lambda i, *_: (0, 0))
    grid_spec = pltpu.PrefetchScalarGridSpec(
        num_scalar_prefetch=3,
        grid=(GRID,),
        in_specs=[
            pl.BlockSpec(memory_space=pl.ANY),
            pl.BlockSpec(memory_space=pl.ANY),
            full((HID, EMB)), full((HID, EMB)), full((EMB, EMB)),
        ],
        out_specs=[pl.BlockSpec((BB, EMB), lambda i, *_: (i, 0))] * 4,
        scratch_shapes=[
            pltpu.VMEM((P, BB, HID), jnp.float32),
            pltpu.VMEM((P, BB, HID), jnp.float32),
            pltpu.VMEM((P, BB, HID), jnp.float32),
            pltpu.SemaphoreType.DMA,
            pltpu.SemaphoreType.DMA,
            pltpu.SemaphoreType.DMA,
        ],
    )
    return pl.pallas_call(
        _fused_body,
        grid_spec=grid_spec,
        out_shape=(out, out, out, out),
    )(users.astype(jnp.int32), pos_items.astype(jnp.int32),
      neg_items.astype(jnp.int32),
      multi_user_embed, multi_item_embed, W_u, W_i, A)
